# baseline (speedup 1.0000x reference)
import jax
import jax.numpy as jnp
from jax.experimental import pallas as pl
from jax.experimental.pallas import tpu as pltpu

N, L, C = 8, 576, 768
_INF = float("inf")


def _msm_kernel(z_ref, out_ref,
                zb2_ref, zb3_ref, b2h_ref, b2hb_ref, a2h_ref,
                h0_ref, h1_ref):
    k = pl.program_id(0)

    @pl.when(k == 0)
    def _prologue():
        ones = jnp.ones((1, C), jnp.float32)
        for r in range(N):
            z = z_ref[r]
            zb = z.astype(jnp.bfloat16)
            zb2_ref[r * L:(r + 1) * L, :] = zb
            zb3_ref[r] = zb
            sq = 0.5 * (z * z)
            b2h = jnp.sum(sq, axis=1, keepdims=True)
            b2h_ref[r * L:(r + 1) * L, :] = b2h
            b2hb_ref[r * L:(r + 1) * L, :] = b2h.astype(jnp.bfloat16)
            a2h_ref[r] = jax.lax.dot_general(
                ones, sq, (((1,), (1,)), ((), ())),
                preferred_element_type=jnp.float32)
        h0_ref[...] = jnp.full((N * L, L), -_INF, jnp.bfloat16)
        h1_ref[...] = jnp.full((N * L, L), -_INF, jnp.bfloat16)

    i_d = jnp.minimum(k, N - 1)
    i_e = jnp.maximum(k - 1, 0)

    def _dot(h_ref):
        h_ref[...] = jax.lax.dot_general(
            zb2_ref[...], zb3_ref[i_d], (((1,), (1,)), ((), ())),
            preferred_element_type=jnp.float32).astype(jnp.bfloat16)

    def _epilogue(h_ref):
        a2h = a2h_ref[i_e]
        m1 = jnp.full((1, L), _INF, jnp.float32)
        m2 = m1
        for j in range(N):
            blk = slice(j * L, (j + 1) * L)
            t = jnp.min(b2hb_ref[blk, :] - h_ref[blk, :],
                        axis=0, keepdims=True)
            v = jnp.maximum(2.0 * (a2h + t.astype(jnp.float32)), 0.0)
            v = jnp.where(j == i_e, _INF, v)
            m2 = jnp.minimum(m2, jnp.maximum(m1, v))
            m1 = jnp.minimum(m1, v)
        out_ref[i_e] = 0.5 * (jnp.sqrt(m1) + jnp.sqrt(m2))

    @pl.when(k % 2 == 0)
    def _even():
        _dot(h0_ref)
        _epilogue(h1_ref)

    @pl.when(k % 2 == 1)
    def _odd():
        _dot(h1_ref)
        _epilogue(h0_ref)


@jax.jit
def kernel(Z):
    full = lambda s: pl.BlockSpec(s, lambda k: (0,) * len(s))
    out = pl.pallas_call(
        _msm_kernel,
        grid=(N + 1,),
        in_specs=[full((N, L, C))],
        out_specs=full((N, 1, L)),
        out_shape=jax.ShapeDtypeStruct((N, 1, L), jnp.float32),
        scratch_shapes=[
            pltpu.VMEM((N * L, C), jnp.bfloat16),
            pltpu.VMEM((N, L, C), jnp.bfloat16),
            pltpu.VMEM((N * L, 1), jnp.float32),
            pltpu.VMEM((N * L, 1), jnp.bfloat16),
            pltpu.VMEM((N, 1, L), jnp.float32),
            pltpu.VMEM((N * L, L), jnp.bfloat16),
            pltpu.VMEM((N * L, L), jnp.bfloat16),
        ],
    )(Z)
    return out[:, 0, :]
